# baseline (speedup 1.0000x reference)
import functools

import jax
import jax.numpy as jnp
from jax import lax
from jax.experimental import pallas as pl
from jax.experimental.pallas import tpu as pltpu
from jax.experimental.pallas import tpu_sc as plsc

D_MODEL = 64
VOCAB = 1_000_000
BATCH = 4096
SEQ = 200
N_IDX = BATCH * SEQ

_info = plsc.get_sparse_core_info()
NUM_WORKERS = _info.num_cores * _info.num_subcores
ROWS_PER_WORKER = BATCH // NUM_WORKERS
ROWS_PER_CHUNK = 4
CHUNK = ROWS_PER_CHUNK * SEQ
NCHUNK = ROWS_PER_WORKER // ROWS_PER_CHUNK
NPAIR = NCHUNK // 2

TR_BW = 8192
TR_GRID = (VOCAB + 2 * TR_BW - 1) // (2 * TR_BW)
NROWS = TR_GRID * TR_BW


def _transpose_body(w_ref, o_ref):
    w = w_ref[...]
    eye = jnp.eye(D_MODEL, dtype=jnp.float32)
    dims = (((0,), (0,)), ((), ()))

    def t(x):
        return lax.dot_general(
            x, eye, dims, preferred_element_type=jnp.float32
        )

    o_ref[...] = jnp.concatenate(
        [t(w[:, :TR_BW]), t(w[:, TR_BW:])], axis=1
    )


def _transpose_table(W_E):
    return pl.pallas_call(
        _transpose_body,
        grid=(TR_GRID,),
        in_specs=[pl.BlockSpec((D_MODEL, 2 * TR_BW), lambda i: (0, i))],
        out_specs=pl.BlockSpec((TR_BW, 2 * D_MODEL), lambda i: (i, 0)),
        out_shape=jax.ShapeDtypeStruct((NROWS, 2 * D_MODEL), jnp.float32),
    )(W_E)


@functools.partial(
    pl.kernel,
    out_type=jax.ShapeDtypeStruct((BATCH, SEQ, D_MODEL), jnp.float32),
    mesh=plsc.VectorSubcoreMesh(core_axis_name="c", subcore_axis_name="s"),
    scratch_types=[
        pltpu.VMEM((CHUNK,), jnp.int32),
        pltpu.VMEM((CHUNK,), jnp.int32),
        pltpu.VMEM((CHUNK,), jnp.int32),
        pltpu.VMEM((CHUNK,), jnp.int32),
        pltpu.VMEM((CHUNK, D_MODEL), jnp.float32),
        pltpu.VMEM((CHUNK, D_MODEL), jnp.float32),
        pltpu.SemaphoreType.DMA,
        pltpu.SemaphoreType.DMA,
    ],
    compiler_params=pltpu.CompilerParams(use_tc_tiling_on_sc=False),
)
def _sc_gather(
    table_hbm,
    idx_hbm,
    out_hbm,
    idx_a,
    idx_b,
    row_a,
    row_b,
    rows_a,
    rows_b,
    sem_a,
    sem_b,
):
    wid = lax.axis_index("s") * _info.num_cores + lax.axis_index("c")
    w_row = wid * ROWS_PER_WORKER

    def remap(idx_v, row_v, row0):
        pltpu.sync_copy(idx_hbm.at[pl.ds(row0 * SEQ, CHUNK)], idx_v)
        for k in range(CHUNK // 16):
            s = pl.ds(k * 16, 16)
            v = idx_v[s]
            b = lax.shift_right_logical(v, 14)
            wm = lax.bitwise_and(v, 8191)
            h = lax.bitwise_and(lax.shift_right_logical(v, 13), 1)
            row_v[s] = lax.bitwise_or(
                lax.bitwise_or(
                    lax.shift_left(b, 14), lax.shift_left(wm, 1)
                ),
                h,
            )

    def start(row_v, rows_v, sem):
        return pltpu.async_copy(table_hbm.at[row_v], rows_v, sem)

    def finish(row_v, rows_v, sem, row0):
        pltpu.make_async_copy(table_hbm.at[row_v], rows_v, sem).wait()
        for r in range(ROWS_PER_CHUNK):
            pltpu.sync_copy(
                rows_v.at[pl.ds(r * SEQ, SEQ)], out_hbm.at[row0 + r]
            )

    remap(idx_a, row_a, w_row)
    start(row_a, rows_a, sem_a)

    def body(cc, carry):
        r0 = w_row + 2 * cc * ROWS_PER_CHUNK
        r1 = r0 + ROWS_PER_CHUNK
        r2 = r1 + ROWS_PER_CHUNK
        remap(idx_b, row_b, r1)
        start(row_b, rows_b, sem_b)
        finish(row_a, rows_a, sem_a, r0)

        @pl.when(cc < NPAIR - 1)
        def _():
            remap(idx_a, row_a, r2)
            start(row_a, rows_a, sem_a)

        finish(row_b, rows_b, sem_b, r1)
        return carry

    lax.fori_loop(0, NPAIR, body, 0)


def kernel(x, W_E):
    W_T2 = _transpose_table(W_E)
    W_T = W_T2.reshape(2 * NROWS, D_MODEL)
    idx = x.reshape(N_IDX).astype(jnp.int32)
    return _sc_gather(W_T, idx)


# speedup vs baseline: 1.0173x; 1.0173x over previous
import functools

import jax
import jax.numpy as jnp
from jax import lax
from jax.experimental import pallas as pl
from jax.experimental.pallas import tpu as pltpu
from jax.experimental.pallas import tpu_sc as plsc

D_MODEL = 64
VOCAB = 1_000_000
BATCH = 4096
SEQ = 200
N_IDX = BATCH * SEQ

_info = plsc.get_sparse_core_info()
NUM_WORKERS = _info.num_cores * _info.num_subcores
ROWS_PER_WORKER = BATCH // NUM_WORKERS
ROWS_PER_CHUNK = 4
CHUNK = ROWS_PER_CHUNK * SEQ
NCHUNK = ROWS_PER_WORKER // ROWS_PER_CHUNK
NPAIR = NCHUNK // 2

TR_BW = 16384
TR_GRID = (VOCAB + 2 * TR_BW - 1) // (2 * TR_BW)
NROWS = TR_GRID * TR_BW
H_SH = TR_BW.bit_length() - 1
B_SH = H_SH + 1


def _transpose_body(w_ref, o_ref):
    w = w_ref[...]
    o_ref[...] = jnp.concatenate(
        [w[:, :TR_BW].T, w[:, TR_BW:].T], axis=1
    )


def _transpose_table(W_E):
    return pl.pallas_call(
        _transpose_body,
        grid=(TR_GRID,),
        in_specs=[pl.BlockSpec((D_MODEL, 2 * TR_BW), lambda i: (0, i))],
        out_specs=pl.BlockSpec((TR_BW, 2 * D_MODEL), lambda i: (i, 0)),
        out_shape=jax.ShapeDtypeStruct((NROWS, 2 * D_MODEL), jnp.float32),
    )(W_E)


@functools.partial(
    pl.kernel,
    out_type=jax.ShapeDtypeStruct((BATCH, SEQ, D_MODEL), jnp.float32),
    mesh=plsc.VectorSubcoreMesh(core_axis_name="c", subcore_axis_name="s"),
    scratch_types=[
        pltpu.VMEM((CHUNK,), jnp.int32),
        pltpu.VMEM((CHUNK,), jnp.int32),
        pltpu.VMEM((CHUNK,), jnp.int32),
        pltpu.VMEM((CHUNK,), jnp.int32),
        pltpu.VMEM((CHUNK, D_MODEL), jnp.float32),
        pltpu.VMEM((CHUNK, D_MODEL), jnp.float32),
        pltpu.SemaphoreType.DMA,
        pltpu.SemaphoreType.DMA,
    ],
    compiler_params=pltpu.CompilerParams(use_tc_tiling_on_sc=False),
)
def _sc_gather(
    table_hbm,
    idx_hbm,
    out_hbm,
    idx_a,
    idx_b,
    row_a,
    row_b,
    rows_a,
    rows_b,
    sem_a,
    sem_b,
):
    wid = lax.axis_index("s") * _info.num_cores + lax.axis_index("c")
    w_row = wid * ROWS_PER_WORKER

    def remap(idx_v, row_v, row0):
        pltpu.sync_copy(idx_hbm.at[pl.ds(row0 * SEQ, CHUNK)], idx_v)
        for k in range(CHUNK // 16):
            s = pl.ds(k * 16, 16)
            v = idx_v[s]
            b = lax.shift_right_logical(v, B_SH)
            wm = lax.bitwise_and(v, TR_BW - 1)
            h = lax.bitwise_and(lax.shift_right_logical(v, H_SH), 1)
            row_v[s] = lax.bitwise_or(
                lax.bitwise_or(
                    lax.shift_left(b, B_SH), lax.shift_left(wm, 1)
                ),
                h,
            )

    def start(row_v, rows_v, sem):
        return pltpu.async_copy(table_hbm.at[row_v], rows_v, sem)

    def finish(row_v, rows_v, sem, row0):
        pltpu.make_async_copy(table_hbm.at[row_v], rows_v, sem).wait()
        for r in range(ROWS_PER_CHUNK):
            pltpu.sync_copy(
                rows_v.at[pl.ds(r * SEQ, SEQ)], out_hbm.at[row0 + r]
            )

    remap(idx_a, row_a, w_row)
    start(row_a, rows_a, sem_a)

    def body(cc, carry):
        r0 = w_row + 2 * cc * ROWS_PER_CHUNK
        r1 = r0 + ROWS_PER_CHUNK
        r2 = r1 + ROWS_PER_CHUNK
        remap(idx_b, row_b, r1)
        start(row_b, rows_b, sem_b)
        finish(row_a, rows_a, sem_a, r0)

        @pl.when(cc < NPAIR - 1)
        def _():
            remap(idx_a, row_a, r2)
            start(row_a, rows_a, sem_a)

        finish(row_b, rows_b, sem_b, r1)
        return carry

    lax.fori_loop(0, NPAIR, body, 0)


def kernel(x, W_E):
    W_T2 = _transpose_table(W_E)
    W_T = W_T2.reshape(2 * NROWS, D_MODEL)
    idx = x.reshape(N_IDX).astype(jnp.int32)
    return _sc_gather(W_T, idx)
